# baseline (speedup 1.0000x reference)
import jax
import jax.numpy as jnp
from jax.experimental import pallas as pl

B = 2048
D = 768
H = 512
E = 8
K = 2


def _moe_block_kernel(x_ref, wr_ref, br_ref, w1_ref, b1_ref, w2_ref, b2_ref,
                      out_ref):
    xb = x_ref[...]
    logits = jax.lax.dot_general(
        xb, wr_ref[...], (((1,), (1,)), ((), ())),
        preferred_element_type=jnp.float32) + br_ref[...]
    m = jnp.max(logits, axis=-1, keepdims=True)
    ex = jnp.exp(logits - m)
    probs = ex / jnp.sum(ex, axis=-1, keepdims=True)

    eids = jax.lax.broadcasted_iota(jnp.int32, logits.shape, 1)
    i1 = jnp.argmax(logits, axis=-1, keepdims=True)
    masked = jnp.where(eids == i1, -jnp.inf, logits)
    i2 = jnp.argmax(masked, axis=-1, keepdims=True)
    sel = (eids == i1) | (eids == i2)
    scale = jnp.where(sel, probs, 0.0)

    bt = xb.shape[0]
    xb16 = xb.astype(jnp.bfloat16)
    h = jax.lax.dot_general(
        xb16, w1_ref[...], (((1,), (1,)), ((), ())),
        preferred_element_type=jnp.float32) + b1_ref[...]
    h = jnp.maximum(h, 0.0)
    h = h.reshape(bt, E, H) * scale[:, :, None]
    h16 = h.reshape(bt, E * H).astype(jnp.bfloat16)
    y = jax.lax.dot_general(
        h16, w2_ref[...], (((1,), (0,)), ((), ())),
        preferred_element_type=jnp.float32)
    y = y + jax.lax.dot_general(
        scale, b2_ref[...], (((1,), (0,)), ((), ())),
        preferred_element_type=jnp.float32)
    out_ref[...] = y


def kernel(x, Wr, br, W1, b1, W2, b2):
    BT = 256
    grid = (B // BT,)
    br2 = br.reshape(1, E)
    w1c = W1.reshape(E * H, D).astype(jnp.bfloat16)
    b1c = b1.reshape(1, E * H)
    w2c = W2.transpose(0, 2, 1).reshape(E * H, D).astype(jnp.bfloat16)
    out = pl.pallas_call(
        _moe_block_kernel,
        grid=grid,
        in_specs=[
            pl.BlockSpec((BT, D), lambda i: (i, 0)),
            pl.BlockSpec((E, D), lambda i: (0, 0)),
            pl.BlockSpec((1, E), lambda i: (0, 0)),
            pl.BlockSpec((E * H, D), lambda i: (0, 0)),
            pl.BlockSpec((1, E * H), lambda i: (0, 0)),
            pl.BlockSpec((E * H, D), lambda i: (0, 0)),
            pl.BlockSpec((E, D), lambda i: (0, 0)),
        ],
        out_specs=pl.BlockSpec((BT, D), lambda i: (i, 0)),
        out_shape=jax.ShapeDtypeStruct((B, D), jnp.float32),
    )(x, Wr, br2, w1c, b1c, w2c, b2)
    return out


# speedup vs baseline: 2.0770x; 2.0770x over previous
import jax
import jax.numpy as jnp
from jax.experimental import pallas as pl
from jax.experimental.pallas import tpu as pltpu

B = 2048
D = 768
H = 512
E = 8
K = 2


def _moe_kernel(x_ref, wr_ref, br_ref, w1_ref, b1_ref, w2_ref, b2_ref,
                out_ref, x16_ref, scale_ref):
    e = pl.program_id(0)

    @pl.when(e == 0)
    def _prologue():
        xb = x_ref[...]
        logits = jax.lax.dot_general(
            xb, wr_ref[...], (((1,), (1,)), ((), ())),
            preferred_element_type=jnp.float32) + br_ref[...]
        m = jnp.max(logits, axis=-1, keepdims=True)
        ex = jnp.exp(logits - m)
        probs = ex / jnp.sum(ex, axis=-1, keepdims=True)
        eids = jax.lax.broadcasted_iota(jnp.int32, logits.shape, 1)
        i1 = jnp.argmax(logits, axis=-1, keepdims=True)
        masked = jnp.where(eids == i1, -jnp.inf, logits)
        i2 = jnp.argmax(masked, axis=-1, keepdims=True)
        sel = (eids == i1) | (eids == i2)
        scale = jnp.where(sel, probs, 0.0)
        scale_ref[...] = scale
        x16_ref[...] = xb.astype(jnp.bfloat16)
        out_ref[...] = jax.lax.dot_general(
            scale, b2_ref[...], (((1,), (0,)), ((), ())),
            preferred_element_type=jnp.float32)

    x16 = x16_ref[...]
    sc = scale_ref[...]
    cols = jax.lax.broadcasted_iota(jnp.int32, sc.shape, 1)
    se = jnp.sum(jnp.where(cols == e, sc, 0.0), axis=1, keepdims=True)
    w1e = w1_ref[0].astype(jnp.bfloat16)
    w2e = w2_ref[0].astype(jnp.bfloat16)
    h = jax.lax.dot_general(
        x16, w1e, (((1,), (1,)), ((), ())),
        preferred_element_type=jnp.float32) + b1_ref[0]
    h = jnp.maximum(h, 0.0)
    h16 = (h * se).astype(jnp.bfloat16)
    out_ref[...] += jax.lax.dot_general(
        h16, w2e, (((1,), (1,)), ((), ())),
        preferred_element_type=jnp.float32)


def kernel(x, Wr, br, W1, b1, W2, b2):
    br2 = br.reshape(1, E)
    b13 = b1.reshape(E, 1, H)
    out = pl.pallas_call(
        _moe_kernel,
        grid=(E,),
        in_specs=[
            pl.BlockSpec((B, D), lambda e: (0, 0)),
            pl.BlockSpec((E, D), lambda e: (0, 0)),
            pl.BlockSpec((1, E), lambda e: (0, 0)),
            pl.BlockSpec((1, H, D), lambda e: (e, 0, 0)),
            pl.BlockSpec((1, 1, H), lambda e: (e, 0, 0)),
            pl.BlockSpec((1, D, H), lambda e: (e, 0, 0)),
            pl.BlockSpec((E, D), lambda e: (0, 0)),
        ],
        out_specs=pl.BlockSpec((B, D), lambda e: (0, 0)),
        out_shape=jax.ShapeDtypeStruct((B, D), jnp.float32),
        scratch_shapes=[
            pltpu.VMEM((B, D), jnp.bfloat16),
            pltpu.VMEM((B, E), jnp.float32),
        ],
    )(x, Wr, br2, W1, b13, W2, b2)
    return out
